# Initial kernel scaffold; baseline (speedup 1.0000x reference)
import functools

import jax
import jax.numpy as jnp
from jax import lax
from jax.experimental import pallas as pl
from jax.experimental.pallas import tpu as pltpu
from jax.experimental.pallas import tpu_sc as plsc

_B_TOTAL = 4096 * 50
_D = 64
_NC, _NS = 2, 16
_NW = _NC * _NS
_CHUNK = 800
_NCHUNK = _B_TOTAL // (_NW * _CHUNK)
_NBUF = 2


def _make_gather():
  mesh = plsc.VectorSubcoreMesh(core_axis_name="c", subcore_axis_name="s")

  @functools.partial(
      pl.kernel,
      mesh=mesh,
      out_type=jax.ShapeDtypeStruct((_B_TOTAL, _D), jnp.float32),
      scratch_types=[
          pltpu.VMEM((_NCHUNK, _CHUNK), jnp.int32),
          pltpu.VMEM((_NBUF, _CHUNK, _D), jnp.float32),
          pltpu.SemaphoreType.DMA,
          pltpu.SemaphoreType.DMA,
      ],
  )
  def gather_kernel(table_hbm, idx_hbm, out_hbm, idx_v, rows_v, gsem, ssem):
    wid = lax.axis_index("s") * _NC + lax.axis_index("c")
    row0 = wid * _NCHUNK * _CHUNK

    pltpu.sync_copy(idx_hbm.at[pl.ds(wid * _NCHUNK, _NCHUNK)], idx_v)

    gathers = [None] * _NCHUNK
    stores = [None] * _NCHUNK
    gathers[0] = pltpu.async_copy(
        table_hbm.at[idx_v.at[0]], rows_v.at[0], gsem)
    for j in range(_NCHUNK):
      gathers[j].wait()
      stores[j] = pltpu.async_copy(
          rows_v.at[j % _NBUF],
          out_hbm.at[pl.ds(row0 + j * _CHUNK, _CHUNK)],
          ssem)
      if j + 1 < _NCHUNK:
        if j >= 1:
          stores[j - 1].wait()
        gathers[j + 1] = pltpu.async_copy(
            table_hbm.at[idx_v.at[j + 1]], rows_v.at[(j + 1) % _NBUF], gsem)
    stores[_NCHUNK - 2].wait()
    stores[_NCHUNK - 1].wait()

  return gather_kernel


_gather = _make_gather()


def kernel(ind, weight):
  flat = ind.reshape(_NW * _NCHUNK, _CHUNK).astype(jnp.int32)
  out = _gather(weight, flat)
  return out.reshape(*ind.shape, weight.shape[1])


# speedup vs baseline: 4.5776x; 4.5776x over previous
import functools

import jax
import jax.numpy as jnp
from jax import lax
from jax.experimental import pallas as pl
from jax.experimental.pallas import tpu as pltpu
from jax.experimental.pallas import tpu_sc as plsc

_B_TOTAL = 4096 * 50
_D = 64
_NC, _NS = 2, 16
_NW = _NC * _NS
_B_PER_W = _B_TOTAL // _NW
_IDX_CHUNK = 128
_NCHUNK = _B_PER_W // _IDX_CHUNK
_CPS = 5
_STEP_ROWS = _CPS * _IDX_CHUNK
_NSTEP = _NCHUNK // _CPS


def _make_gather():
  mesh = plsc.VectorSubcoreMesh(core_axis_name="c", subcore_axis_name="s")

  @functools.partial(
      pl.kernel,
      mesh=mesh,
      out_type=jax.ShapeDtypeStruct((_B_TOTAL, _D), jnp.float32),
      compiler_params=pltpu.CompilerParams(use_tc_tiling_on_sc=False),
      scratch_types=[
          pltpu.VMEM((_B_PER_W,), jnp.int32),
          pltpu.VMEM((_STEP_ROWS, _D), jnp.float32),
          pltpu.VMEM((_STEP_ROWS, _D), jnp.float32),
          pltpu.SemaphoreType.DMA,
          pltpu.SemaphoreType.DMA,
          pltpu.SemaphoreType.DMA,
          pltpu.SemaphoreType.DMA,
      ],
  )
  def gather_kernel(table_hbm, idx_hbm, out_hbm, idx_v, buf_a, buf_b,
                    gsem_a, gsem_b, ssem_a, ssem_b):
    wid = lax.axis_index("s") * _NC + lax.axis_index("c")
    row0 = pl.multiple_of(wid * _B_PER_W, _STEP_ROWS)

    pltpu.sync_copy(
        idx_hbm.at[pl.ds(pl.multiple_of(wid * _B_PER_W, 8), _B_PER_W)],
        idx_v)

    def fire_step(s, buf, gsem):
      for c in range(_CPS):
        off = pl.multiple_of((s * _CPS + c) * _IDX_CHUNK, _IDX_CHUNK)
        pltpu.async_copy(
            table_hbm.at[idx_v.at[pl.ds(off, _IDX_CHUNK)]],
            buf.at[pl.ds(c * _IDX_CHUNK, _IDX_CHUNK)],
            gsem)

    def drain_gathers(buf, gsem):
      pltpu.make_async_copy(
          table_hbm.at[pl.ds(0, _STEP_ROWS)], buf, gsem).wait()

    def fire_store(s, buf, ssem):
      off = pl.multiple_of(row0 + s * _STEP_ROWS, _STEP_ROWS)
      pltpu.async_copy(buf, out_hbm.at[pl.ds(off, _STEP_ROWS)], ssem)

    def drain_store(buf, ssem):
      pltpu.make_async_copy(
          buf, out_hbm.at[pl.ds(0, _STEP_ROWS)], ssem).wait()

    fire_step(0, buf_a, gsem_a)
    fire_step(1, buf_b, gsem_b)

    def body(i, carry):
      s0 = 2 * i
      drain_gathers(buf_a, gsem_a)
      fire_store(s0, buf_a, ssem_a)
      drain_gathers(buf_b, gsem_b)
      fire_store(s0 + 1, buf_b, ssem_b)

      @pl.when(i < _NSTEP // 2 - 1)
      def _():
        drain_store(buf_a, ssem_a)
        fire_step(s0 + 2, buf_a, gsem_a)
        drain_store(buf_b, ssem_b)
        fire_step(s0 + 3, buf_b, gsem_b)

      return carry

    lax.fori_loop(0, _NSTEP // 2, body, 0)
    drain_store(buf_a, ssem_a)
    drain_store(buf_b, ssem_b)

  return gather_kernel


_gather = _make_gather()


def kernel(ind, weight):
  flat = ind.reshape(_B_TOTAL).astype(jnp.int32)
  out = _gather(weight, flat)
  return out.reshape(*ind.shape, weight.shape[1])


# speedup vs baseline: 4.6766x; 1.0216x over previous
import functools

import jax
import jax.numpy as jnp
from jax import lax
from jax.experimental import pallas as pl
from jax.experimental.pallas import tpu as pltpu
from jax.experimental.pallas import tpu_sc as plsc

_B_TOTAL = 4096 * 50
_D = 64
_NC, _NS = 2, 16
_NW = _NC * _NS
_B_PER_W = _B_TOTAL // _NW
_IDX_CHUNK = 128
_CPS = 2
_STEP_ROWS = _CPS * _IDX_CHUNK
_NSTEP = _B_PER_W // _STEP_ROWS
_NBUF = 5


def _make_gather():
  mesh = plsc.VectorSubcoreMesh(core_axis_name="c", subcore_axis_name="s")

  @functools.partial(
      pl.kernel,
      mesh=mesh,
      out_type=jax.ShapeDtypeStruct((_B_TOTAL, _D), jnp.float32),
      compiler_params=pltpu.CompilerParams(use_tc_tiling_on_sc=False),
      scratch_types=(
          [pltpu.VMEM((_B_PER_W,), jnp.int32)]
          + [pltpu.VMEM((_STEP_ROWS, _D), jnp.float32)] * _NBUF
          + [pltpu.SemaphoreType.DMA] * (2 * _NBUF)
      ),
  )
  def gather_kernel(table_hbm, idx_hbm, out_hbm, idx_v, *scratch):
    bufs = scratch[:_NBUF]
    gsems = scratch[_NBUF:2 * _NBUF]
    ssems = scratch[2 * _NBUF:]
    wid = lax.axis_index("s") * _NC + lax.axis_index("c")
    row0 = pl.multiple_of(wid * _B_PER_W, _STEP_ROWS)

    pltpu.sync_copy(idx_hbm.at[pl.ds(row0, _B_PER_W)], idx_v)

    def step_gathers(t, b):
      for c in range(_CPS):
        off = pl.multiple_of((t * _CPS + c) * _IDX_CHUNK, _IDX_CHUNK)
        yield pltpu.make_async_copy(
            table_hbm.at[idx_v.at[pl.ds(off, _IDX_CHUNK)]],
            bufs[b].at[pl.ds(c * _IDX_CHUNK, _IDX_CHUNK)],
            gsems[b])

    def fire_step(t, b):
      for cp in step_gathers(t, b):
        cp.start()

    def drain_gathers(t, b):
      for cp in step_gathers(t, b):
        cp.wait()

    def store_copy(s, b):
      off = pl.multiple_of(row0 + s * _STEP_ROWS, _STEP_ROWS)
      return pltpu.make_async_copy(
          bufs[b], out_hbm.at[pl.ds(off, _STEP_ROWS)], ssems[b])

    def fire_store(s, b):
      store_copy(s, b).start()

    def drain_store(s, b):
      store_copy(s, b).wait()

    for t in range(_NBUF - 1):
      fire_step(t, t)

    def body(i, carry):
      for b in range(_NBUF):
        s = i * _NBUF + b
        drain_gathers(s, b)
        fire_store(s, b)
        t = s + _NBUF - 1
        tb = (b + _NBUF - 1) % _NBUF
        if b == 0:
          @pl.when(i >= 1)
          def _(t=t, tb=tb):
            drain_store(t - _NBUF, tb)
          fire_step(t, tb)
        else:
          @pl.when(t < _NSTEP)
          def _(t=t, tb=tb):
            drain_store(t - _NBUF, tb)
            fire_step(t, tb)
      return carry

    lax.fori_loop(0, _NSTEP // _NBUF, body, 0)
    for b in range(_NBUF):
      drain_store(_NSTEP - _NBUF + b, b)

  return gather_kernel


_gather = _make_gather()


def kernel(ind, weight):
  flat = ind.reshape(_B_TOTAL).astype(jnp.int32)
  out = _gather(weight, flat)
  return out.reshape(*ind.shape, weight.shape[1])


# speedup vs baseline: 7.1613x; 1.5313x over previous
import functools

import jax
import jax.numpy as jnp
from jax import lax
from jax.experimental import pallas as pl
from jax.experimental.pallas import tpu as pltpu
from jax.experimental.pallas import tpu_sc as plsc

_B = 4096
_T = 50
_D = 64
_NC, _NS = 2, 16
_NW = _NC * _NS
_BPW = _B // _NW
_TP = _T // 2
_NG = 3
_NFULL = (_T - 2) // _NG


def _make_sc_gather():
  mesh = plsc.VectorSubcoreMesh(core_axis_name="c", subcore_axis_name="s")

  @functools.partial(
      pl.kernel,
      mesh=mesh,
      out_type=jax.ShapeDtypeStruct((_TP, _B, 2 * _D), jnp.float32),
      compiler_params=pltpu.CompilerParams(use_tc_tiling_on_sc=False),
      scratch_types=(
          [pltpu.VMEM((_T, _BPW), jnp.int32)]
          + [pltpu.VMEM((_BPW, _D), jnp.float32)] * _NG
          + [pltpu.SemaphoreType.DMA] * (2 * _NG)
      ),
  )
  def sc_gather(indt_hbm, table_hbm, inter_hbm, idx_v, *scratch):
    gbufs = scratch[:_NG]
    gsems = scratch[_NG:2 * _NG]
    ssems = scratch[2 * _NG:]
    wid = lax.axis_index("s") * _NC + lax.axis_index("c")
    woff = pl.multiple_of(wid * _BPW, _BPW)

    pltpu.sync_copy(indt_hbm.at[:, pl.ds(woff, _BPW)], idx_v)

    def gather_copy(t, g):
      return pltpu.make_async_copy(
          table_hbm.at[idx_v.at[t]], gbufs[g], gsems[g])

    def store_copy(t, g):
      tp = lax.rem(t, _TP)
      par = t // _TP
      return pltpu.make_async_copy(
          gbufs[g],
          inter_hbm.at[tp, pl.ds(woff, _BPW),
                       pl.ds(pl.multiple_of(par * _D, _D), _D)],
          ssems[g])

    gather_copy(0, 0).start()
    gather_copy(1, 1).start()

    def body(o, carry):
      for k in range(_NG):
        t = o * _NG + k
        kn = (k + 2) % _NG
        gather_copy(t, k).wait()
        if k == 0:
          @pl.when(o >= 1)
          def _(t=t, kn=kn):
            store_copy(t - 1, kn).wait()
        else:
          store_copy(t - 1, kn).wait()
        gather_copy(t + 2, kn).start()
        store_copy(t, k).start()
      return carry
    lax.fori_loop(0, _NFULL, body, 0)

    gather_copy(_T - 2, 0).wait()
    store_copy(_T - 3, 2).wait()
    store_copy(_T - 2, 0).start()
    gather_copy(_T - 1, 1).wait()
    store_copy(_T - 1, 1).start()
    store_copy(_T - 2, 0).wait()
    store_copy(_T - 1, 1).wait()

  return sc_gather


_sc_gather = _make_sc_gather()


def _tc_body(x_ref, o_ref):
  xt = x_ref[0].T
  o_ref[0, 0] = xt[:_D, :]
  o_ref[1, 0] = xt[_D:, :]


_tc_transpose = pl.pallas_call(
    _tc_body,
    grid=(_TP,),
    in_specs=[pl.BlockSpec((1, _B, 2 * _D), lambda tp: (tp, 0, 0))],
    out_specs=pl.BlockSpec((2, 1, _D, _B), lambda tp: (0, tp, 0, 0)),
    out_shape=jax.ShapeDtypeStruct((2, _TP, _D, _B), jnp.float32),
)


def kernel(ind, weight):
  inter = _sc_gather(ind.astype(jnp.int32).T, weight)
  out4 = _tc_transpose(inter)
  return out4.transpose(3, 0, 1, 2).reshape(_B, _T, _D)


# speedup vs baseline: 7.2625x; 1.0141x over previous
import functools

import jax
import jax.numpy as jnp
from jax import lax
from jax.experimental import pallas as pl
from jax.experimental.pallas import tpu as pltpu
from jax.experimental.pallas import tpu_sc as plsc

_B = 4096
_T = 50
_D = 64
_NC, _NS = 2, 16
_NW = _NC * _NS
_BPW = _B // _NW
_TP = _T // 2
_NG = 4
_NFULL = (_T - 2) // _NG


def _make_sc_gather():
  mesh = plsc.VectorSubcoreMesh(core_axis_name="c", subcore_axis_name="s")

  @functools.partial(
      pl.kernel,
      mesh=mesh,
      out_type=jax.ShapeDtypeStruct((_TP, _B, 2 * _D), jnp.float32),
      compiler_params=pltpu.CompilerParams(use_tc_tiling_on_sc=False),
      scratch_types=(
          [pltpu.VMEM((_T, _BPW), jnp.int32)]
          + [pltpu.VMEM((_BPW, _D), jnp.float32)] * _NG
          + [pltpu.SemaphoreType.DMA] * (2 * _NG)
      ),
  )
  def sc_gather(indt_hbm, table_hbm, inter_hbm, idx_v, *scratch):
    gbufs = scratch[:_NG]
    gsems = scratch[_NG:2 * _NG]
    ssems = scratch[2 * _NG:]
    wid = lax.axis_index("s") * _NC + lax.axis_index("c")
    woff = pl.multiple_of(wid * _BPW, _BPW)

    pltpu.sync_copy(indt_hbm.at[:, pl.ds(woff, _BPW)], idx_v)

    def gather_copy(t, g):
      return pltpu.make_async_copy(
          table_hbm.at[idx_v.at[t]], gbufs[g], gsems[g])

    def store_copy(t, g):
      tp = lax.rem(t, _TP)
      par = t // _TP
      return pltpu.make_async_copy(
          gbufs[g],
          inter_hbm.at[tp, pl.ds(woff, _BPW),
                       pl.ds(pl.multiple_of(par * _D, _D), _D)],
          ssems[g])

    for t0 in range(_NG - 1):
      gather_copy(t0, t0).start()

    def body(o, carry):
      for k in range(_NG):
        t = o * _NG + k
        kn = (k + _NG - 1) % _NG
        gather_copy(t, k).wait()
        if k == 0:
          @pl.when(o >= 1)
          def _(t=t, kn=kn):
            store_copy(t - 1, kn).wait()
        else:
          store_copy(t - 1, kn).wait()
        if k == _NG - 1:
          @pl.when(o < _NFULL - 1)
          def _(t=t, kn=kn):
            gather_copy(t + _NG - 1, kn).start()
        else:
          gather_copy(t + _NG - 1, kn).start()
        store_copy(t, k).start()
      return carry
    lax.fori_loop(0, _NFULL, body, 0)

    gather_copy(_T - 2, (_T - 2) % _NG).wait()
    store_copy(_T - 3, (_T - 3) % _NG).wait()
    store_copy(_T - 2, (_T - 2) % _NG).start()
    gather_copy(_T - 1, (_T - 1) % _NG).wait()
    store_copy(_T - 1, (_T - 1) % _NG).start()
    store_copy(_T - 2, (_T - 2) % _NG).wait()
    store_copy(_T - 1, (_T - 1) % _NG).wait()

  return sc_gather


_sc_gather = _make_sc_gather()


def _tc_body(x_ref, o_ref):
  xt = x_ref[0].T
  o_ref[0, 0] = xt[:_D, :]
  o_ref[1, 0] = xt[_D:, :]


_tc_transpose = pl.pallas_call(
    _tc_body,
    grid=(_TP,),
    in_specs=[pl.BlockSpec((1, _B, 2 * _D), lambda tp: (tp, 0, 0))],
    out_specs=pl.BlockSpec((2, 1, _D, _B), lambda tp: (0, tp, 0, 0)),
    out_shape=jax.ShapeDtypeStruct((2, _TP, _D, _B), jnp.float32),
)


def kernel(ind, weight):
  inter = _sc_gather(ind.astype(jnp.int32).T, weight)
  out4 = _tc_transpose(inter)
  return out4.transpose(3, 0, 1, 2).reshape(_B, _T, _D)
